# Initial kernel scaffold; baseline (speedup 1.0000x reference)
import jax
import jax.numpy as jnp
from jax.experimental import pallas as pl


def kernel(x, table):
    raise NotImplementedError("write your pallas kernel here")


# speedup vs baseline: 4.3173x; 4.3173x over previous
import functools
import math

import jax
import jax.numpy as jnp
from jax import lax
from jax.experimental import pallas as pl
from jax.experimental.pallas import tpu as pltpu
from jax.experimental.pallas import tpu_sc as plsc

VOCAB = 1000000
D = 32
SCALE = math.sqrt(D)

NC = 2
NS = 16
NW = NC * NS

B_TOTAL = 16384 * 200
PER_W = B_TOTAL // NW
CHUNK = 1024
NCHUNK = PER_W // CHUNK
SUB = 128
NSUB = CHUNK // SUB
VECS_PER_CHUNK = CHUNK * D // 16


def _emb_body(x_hbm, table_hbm, out_hbm, idx_v, rows_v, sem0, sem1):
    wid = lax.axis_index("s") * NC + lax.axis_index("c")
    base = wid * PER_W
    sems = (sem0, sem1)

    def load_and_fire(g, p):
        start = base + g * CHUNK
        pltpu.sync_copy(x_hbm.at[pl.ds(start, CHUNK)], idx_v.at[p])
        for s in range(NSUB):
            pltpu.async_copy(
                table_hbm.at[idx_v.at[p, pl.ds(s * SUB, SUB)]],
                rows_v.at[p, pl.ds(s * SUB, SUB)],
                sems[p],
            )

    def drain(p):
        for s in range(NSUB):
            pltpu.make_async_copy(
                table_hbm.at[idx_v.at[p, pl.ds(s * SUB, SUB)]],
                rows_v.at[p, pl.ds(s * SUB, SUB)],
                sems[p],
            ).wait()

    def scale_and_store(g, p):
        @pl.loop(0, CHUNK)
        def _scale(j):
            rows_v[p, j, pl.ds(0, 16)] = rows_v[p, j, pl.ds(0, 16)] * SCALE
            rows_v[p, j, pl.ds(16, 16)] = rows_v[p, j, pl.ds(16, 16)] * SCALE

        pltpu.sync_copy(rows_v.at[p], out_hbm.at[pl.ds(base + g * CHUNK, CHUNK)])

    load_and_fire(0, 0)

    @pl.loop(0, NCHUNK, step=2)
    def _chunks(g):
        for b in range(2):
            gb = g + b

            @pl.when(gb + 1 < NCHUNK)
            def _prefetch():
                load_and_fire(gb + 1, 1 - b)

            drain(b)
            scale_and_store(gb, b)


@functools.partial(jax.jit, static_argnames=())
def _emb(x_flat, table):
    mesh = plsc.VectorSubcoreMesh(core_axis_name="c", subcore_axis_name="s")
    run = pl.kernel(
        _emb_body,
        out_type=jax.ShapeDtypeStruct((B_TOTAL, D), jnp.float32),
        mesh=mesh,
        scratch_types=[
            pltpu.VMEM((2, CHUNK), jnp.int32),
            pltpu.VMEM((2, CHUNK, D), jnp.float32),
            pltpu.SemaphoreType.DMA,
            pltpu.SemaphoreType.DMA,
        ],
        name="sc_text_embedding",
        compiler_params=pltpu.CompilerParams(use_tc_tiling_on_sc=False),
    )
    return run(x_flat, table)


def kernel(x, table):
    x_flat = x.reshape(-1).astype(jnp.int32)
    out = _emb(x_flat, table)
    return out.reshape(x.shape[0], x.shape[1], D)
